# baseline (speedup 1.0000x reference)
import functools

import jax
import jax.numpy as jnp
from jax import lax
from jax.experimental import pallas as pl
from jax.experimental.pallas import tpu as pltpu
from jax.experimental.pallas import tpu_sc as plsc

_P = 128
_BR = _P + 8
_BC = 2 * _P
_H = 1024
_B = 256
_CB = 8
_NTILES = 32
_BAND = _H // _NTILES
_L = 16
_PW = _P + 2 * _L
_RB = 16
_SLICE = (_P * _P) // 16
_SROWS = _SLICE // _P
_BTC = 96
_NB = 8


def _pn_body(waves_ref, pn_ref):
    i = pl.program_id(0)

    @pl.when(i == 0)
    def _init():
        pn_ref[...] = jnp.zeros_like(pn_ref)

    w = waves_ref[:, 0]
    pn_ref[...] += jnp.sum(w * w, axis=0)


def _crop_mul_body(pos_ref, obj_ref, waves_ref, out_ref):
    i = pl.program_id(0)
    for k in range(_CB):
        b = i * _CB + k
        r = pos_ref[b, 0]
        c = pos_ref[b, 1]
        r8 = (r // 8) * 8
        t = r - r8
        c128 = (c // _P) * _P
        s = c - c128
        big = obj_ref[0, pl.ds(r8, _BR), pl.ds(c128, _BC)]
        rolled = pltpu.roll(pltpu.roll(big, _BR - t, axis=0), _BC - s, axis=1)
        out_ref[k, 0] = waves_ref[k, 0] * rolled[: _P, : _P]


def _sc_body(waves_hbm, pntc_hbm, posrc_hbm, rng_hbm, on_hbm,
             wbuf_v, acc_v, pn_v, pntc_v, posrc_v, rng_v, band_v, pn_sh, sem):
    cid = lax.axis_index("c")
    sid = lax.axis_index("s")
    wid = sid * 2 + cid
    y0 = wid * _BAND

    pltpu.sync_copy(posrc_hbm, posrc_v)
    pltpu.sync_copy(rng_hbm, rng_v)
    pltpu.sync_copy(
        pntc_hbm.at[pl.ds(pl.multiple_of(_SROWS * sid, _SROWS), _SROWS)],
        pntc_v,
    )

    zeros = jnp.zeros((_L,), jnp.float32)

    for j in range(_SLICE // _L):
        acc_v[pl.ds(j * _L, _L)] = zeros

    row0 = _SROWS * sid
    nrounds = (_B - _BTC) // _RB

    def _issue(rd, slot):
        return pltpu.async_copy(
            waves_hbm.at[
                pl.ds(pl.multiple_of(_BTC + rd * _RB, _RB), _RB),
                0,
                pl.ds(pl.multiple_of(row0, _SROWS), _SROWS),
                pl.ds(0, _P),
            ],
            wbuf_v.at[slot],
            sem,
        )

    descs = {0: _issue(0, 0)}
    for rd in range(nrounds):
        cur = rd % 2
        if rd + 1 < nrounds:
            descs[rd + 1] = _issue(rd + 1, 1 - cur)
        descs[rd].wait()

        def jr_body(jr, _, cur=cur):
            for jc in range(_P // _L):
                off = jr * _P + jc * _L
                a = acc_v[pl.ds(off, _L)]
                for b in range(_RB):
                    w = wbuf_v[cur, b, jr, pl.ds(jc * _L, _L)]
                    a = a + w * w
                acc_v[pl.ds(off, _L)] = a
            return 0

        lax.fori_loop(0, _SROWS, jr_body, 0)

    for jr in range(_SROWS):
        for jc in range(_P // _L):
            off = jr * _P + jc * _L
            acc_v[pl.ds(off, _L)] += pntc_v[jr, pl.ds(jc * _L, _L)]

    pltpu.sync_copy(
        acc_v,
        pn_sh.at[pl.ds(pl.multiple_of(sid * _SLICE, _SLICE), _SLICE)],
    )

    def zrow(i, _):
        for j in range(_H // _L):
            band_v[i, pl.ds(j * _L, _L)] = zeros
        return 0
    lax.fori_loop(0, _BAND, zrow, 0)

    plsc.subcore_barrier()
    pltpu.sync_copy(pn_sh, pn_v)

    lane = lax.iota(jnp.int32, _L)
    rv = rng_v[wid]
    bstart = rv[0]
    bend = rv[1]

    def pos_body(b, _):
        v = posrc_v[b]
        p0 = v[0]
        p1 = v[1]
        lo = jnp.maximum(p0, y0)
        hi = jnp.minimum(p0 + _P, y0 + _BAND)
        jlo = p1 // _L
        rels = [(jlo * _L - p1 + jj * _L) + lane for jj in range(9)]
        cols = [jnp.clip(rel, 0, _P - 1) for rel in rels]
        masks = [(rel >= 0) & (rel < _P) for rel in rels]

        @plsc.parallel_loop(lo, hi, unroll=2)
        def _row_body(y, p0=p0, jlo=jlo, cols=cols, masks=masks):
            rb = (y - p0) * _P
            yloc = y - y0
            for jj in range(9):
                val = plsc.load_gather(pn_v, [rb + cols[jj]])
                val = jnp.where(masks[jj], val, 0.0)
                cs = pl.multiple_of((jlo + jj) * _L, _L)
                plsc.addupdate(band_v.at[yloc, pl.ds(cs, _L)], val)
        return 0

    lax.fori_loop(bstart, bend, pos_body, 0)
    pltpu.sync_copy(band_v, on_hbm.at[pl.ds(y0, _BAND)])


def _sc_object_norm(waves_flat, pntc, posrc, rngs):
    mesh = plsc.VectorSubcoreMesh(
        core_axis_name="c", subcore_axis_name="s", num_cores=2, num_subcores=16
    )
    return pl.kernel(
        _sc_body,
        out_type=jax.ShapeDtypeStruct((_H, _H), jnp.float32),
        mesh=mesh,
        compiler_params=pltpu.CompilerParams(needs_layout_passes=False),
        scratch_types=[
            pltpu.VMEM((2, _RB, _SROWS, _P), jnp.float32),
            pltpu.VMEM((_SLICE,), jnp.float32),
            pltpu.VMEM((_P * _P,), jnp.float32),
            pltpu.VMEM((_SROWS, _P), jnp.float32),
            pltpu.VMEM((_B, _L), jnp.int32),
            pltpu.VMEM((_NTILES, _L), jnp.int32),
            pltpu.VMEM((_BAND, _H), jnp.float32),
            pltpu.VMEM_SHARED((_P * _P,), jnp.float32),
            pltpu.SemaphoreType.DMA,
        ],
    )(waves_flat, pntc, posrc, rngs)


@jax.jit
def kernel(obj, waves, pos):
    B = waves.shape[0]
    h, w = waves.shape[-2], waves.shape[-1]
    H, W = obj.shape[-2], obj.shape[-1]
    pos32 = pos.astype(jnp.int32)

    order = jnp.argsort(pos32[:, 0])
    pos_sorted = pos32[order]
    p0s = pos_sorted[:, 0]
    band_lo = jnp.arange(_NTILES, dtype=jnp.int32) * _BAND
    starts = jnp.searchsorted(p0s, band_lo - (h - 1), side="left")
    ends = jnp.searchsorted(p0s, band_lo + (_BAND - 1), side="right")
    rngs = jnp.stack(
        [starts.astype(jnp.int32), ends.astype(jnp.int32)], axis=1
    )
    rngs = jnp.pad(rngs, ((0, 0), (0, _L - 2)))
    posrc = jnp.pad(pos_sorted, ((0, 0), (0, _L - 2)))

    pn_tc = pl.pallas_call(
        _pn_body,
        grid=(_BTC // _NB,),
        in_specs=[pl.BlockSpec((_NB, 1, h, w), lambda i: (i, 0, 0, 0))],
        out_specs=pl.BlockSpec((h, w), lambda i: (0, 0)),
        out_shape=jax.ShapeDtypeStruct((h, w), jnp.float32),
    )(waves[:_BTC])

    object_norm = _sc_object_norm(waves, pn_tc, posrc, rngs)

    grid_spec = pltpu.PrefetchScalarGridSpec(
        num_scalar_prefetch=1,
        grid=(B // _CB,),
        in_specs=[
            pl.BlockSpec((1, H, W), lambda i, p: (0, 0, 0)),
            pl.BlockSpec((_CB, 1, h, w), lambda i, p: (i, 0, 0, 0)),
        ],
        out_specs=pl.BlockSpec((_CB, 1, h, w), lambda i, p: (i, 0, 0, 0)),
    )
    out = pl.pallas_call(
        _crop_mul_body,
        grid_spec=grid_spec,
        out_shape=jax.ShapeDtypeStruct((B, 1, h, w), jnp.float32),
    )(pos32, obj, waves)

    return (out, object_norm)


# speedup vs baseline: 1.1155x; 1.1155x over previous
import functools

import jax
import jax.numpy as jnp
from jax import lax
from jax.experimental import pallas as pl
from jax.experimental.pallas import tpu as pltpu
from jax.experimental.pallas import tpu_sc as plsc

_P = 128
_BR = _P + 8
_BC = 2 * _P
_H = 1024
_NB = 8
_CB = 8
_NTILES = 32
_BAND = _H // _NTILES
_L = 16
_PW = _P + 2 * _L


def _pn_body(waves_ref, pn_ref):
    i = pl.program_id(0)

    @pl.when(i == 0)
    def _init():
        pn_ref[...] = jnp.zeros_like(pn_ref)

    w = waves_ref[:, 0]
    pn_ref[...] += jnp.sum(w * w, axis=0)


def _crop_mul_body(pos_ref, obj_ref, waves_ref, out_ref):
    i = pl.program_id(0)
    for k in range(_CB):
        b = i * _CB + k
        r = pos_ref[b, 0]
        c = pos_ref[b, 1]
        r8 = (r // 8) * 8
        t = r - r8
        c128 = (c // _P) * _P
        s = c - c128
        big = obj_ref[0, pl.ds(r8, _BR), pl.ds(c128, _BC)]
        rolled = pltpu.roll(pltpu.roll(big, _BR - t, axis=0), _BC - s, axis=1)
        out_ref[k, 0] = waves_ref[k, 0] * rolled[: _P, : _P]


def _sc_scatter_body(pnp_hbm, posrc_hbm, rng_hbm, on_hbm,
                     pnp_v, posrc_v, rng_v, band_v):
    cid = lax.axis_index("c")
    sid = lax.axis_index("s")
    wid = sid * 2 + cid
    y0 = wid * _BAND

    pltpu.sync_copy(pnp_hbm, pnp_v)
    pltpu.sync_copy(posrc_hbm, posrc_v)
    pltpu.sync_copy(rng_hbm, rng_v)

    zeros = jnp.zeros((_L,), jnp.float32)

    def zrow(i, _):
        for j in range(_H // _L):
            band_v[i, pl.ds(j * _L, _L)] = zeros
        return 0
    lax.fori_loop(0, _BAND, zrow, 0)

    lane = lax.iota(jnp.int32, _L)
    rv = rng_v[wid]
    bstart = rv[0]
    bend = rv[1]

    def pos_body(b, _):
        v = posrc_v[b]
        p0 = v[0]
        p1 = v[1]
        lo = jnp.maximum(p0, y0)
        hi = jnp.minimum(p0 + _P, y0 + _BAND)
        jlo = p1 // _L
        idxc = [(_L + jlo * _L - p1 + jj * _L) + lane for jj in range(9)]

        @plsc.parallel_loop(lo, hi, unroll=2)
        def _row_body(y, p0=p0, jlo=jlo, idxc=idxc):
            rb = (y - p0) * _PW
            yloc = y - y0
            for jj in range(9):
                val = plsc.load_gather(pnp_v, [rb + idxc[jj]])
                cs = pl.multiple_of((jlo + jj) * _L, _L)
                plsc.addupdate(band_v.at[yloc, pl.ds(cs, _L)], val)
        return 0

    lax.fori_loop(bstart, bend, pos_body, 0)
    pltpu.sync_copy(band_v, on_hbm.at[pl.ds(y0, _BAND)])


def _sc_scatter(pnp_flat, posrc, rngs):
    mesh = plsc.VectorSubcoreMesh(
        core_axis_name="c", subcore_axis_name="s", num_cores=2, num_subcores=16
    )
    return pl.kernel(
        _sc_scatter_body,
        out_type=jax.ShapeDtypeStruct((_H, _H), jnp.float32),
        mesh=mesh,
        compiler_params=pltpu.CompilerParams(needs_layout_passes=False),
        scratch_types=[
            pltpu.VMEM((_P * _PW,), jnp.float32),
            pltpu.VMEM((256, _L), jnp.int32),
            pltpu.VMEM((_NTILES, _L), jnp.int32),
            pltpu.VMEM((_BAND, _H), jnp.float32),
        ],
    )(pnp_flat, posrc, rngs)


@jax.jit
def kernel(obj, waves, pos):
    B = waves.shape[0]
    h, w = waves.shape[-2], waves.shape[-1]
    H, W = obj.shape[-2], obj.shape[-1]
    pos32 = pos.astype(jnp.int32)

    patch_norm = pl.pallas_call(
        _pn_body,
        grid=(B // _NB,),
        in_specs=[pl.BlockSpec((_NB, 1, h, w), lambda i: (i, 0, 0, 0))],
        out_specs=pl.BlockSpec((h, w), lambda i: (0, 0)),
        out_shape=jax.ShapeDtypeStruct((h, w), jnp.float32),
    )(waves)

    pn_padded = jnp.pad(patch_norm, ((0, 0), (_L, _L)))

    order = jnp.argsort(pos32[:, 0])
    pos_sorted = pos32[order]
    p0s = pos_sorted[:, 0]
    band_lo = jnp.arange(_NTILES, dtype=jnp.int32) * _BAND
    starts = jnp.searchsorted(p0s, band_lo - (h - 1), side="left")
    ends = jnp.searchsorted(p0s, band_lo + (_BAND - 1), side="right")
    rngs = jnp.stack(
        [starts.astype(jnp.int32), ends.astype(jnp.int32)], axis=1
    )
    rngs = jnp.pad(rngs, ((0, 0), (0, _L - 2)))
    posrc = jnp.pad(pos_sorted, ((0, 0), (0, _L - 2)))

    object_norm = _sc_scatter(pn_padded.reshape(h * _PW), posrc, rngs)

    grid_spec = pltpu.PrefetchScalarGridSpec(
        num_scalar_prefetch=1,
        grid=(B // _CB,),
        in_specs=[
            pl.BlockSpec((1, H, W), lambda i, p: (0, 0, 0)),
            pl.BlockSpec((_CB, 1, h, w), lambda i, p: (i, 0, 0, 0)),
        ],
        out_specs=pl.BlockSpec((_CB, 1, h, w), lambda i, p: (i, 0, 0, 0)),
    )
    out = pl.pallas_call(
        _crop_mul_body,
        grid_spec=grid_spec,
        out_shape=jax.ShapeDtypeStruct((B, 1, h, w), jnp.float32),
    )(pos32, obj, waves)

    return (out, object_norm)
